# baseline (speedup 1.0000x reference)
import functools

import jax
import jax.numpy as jnp
from jax import lax
from jax.experimental import pallas as pl
from jax.experimental.pallas import tpu as pltpu
from jax.experimental.pallas import tpu_sc as plsc

_N = 10000
_E = 320000
_H = 128
_I = 16

_NC = 2
_NS = 16
_NW = _NC * _NS
_EH0 = 163840
_EH1 = _E - _EH0
_NP = 10240
_NROWS = _NP // _NS

_mesh = functools.partial(
    plsc.VectorSubcoreMesh, core_axis_name="c", subcore_axis_name="s"
)


def _silu(x):
    h = x * 0.5
    return h * (1.0 + jnp.tanh(h))


def _sigmoid(x):
    return 0.5 * (1.0 + jnp.tanh(x * 0.5))


def _chunking(epw):
    ch = 128 if epw % 128 == 0 else 80
    return ch, epw // ch


def _proj_body(x_ref, wab_ref, ps_ref, pr_ref):
    ps_ref[...] = jnp.dot(x_ref[0], wab_ref[0], preferred_element_type=jnp.float32)
    pr_ref[...] = jnp.dot(x_ref[1], wab_ref[1], preferred_element_type=jnp.float32)


def _project_nodes(x, wab):
    nb = 2000
    return pl.pallas_call(
        _proj_body,
        grid=(_N // nb,),
        in_specs=[
            pl.BlockSpec((2, nb, _H), lambda i: (0, i, 0)),
            pl.BlockSpec((2, _H, _H), lambda i: (0, 0, 0)),
        ],
        out_specs=[
            pl.BlockSpec((nb, _H), lambda i: (i, 0)),
            pl.BlockSpec((nb, _H), lambda i: (i, 0)),
        ],
        out_shape=[
            jax.ShapeDtypeStruct((_N, _H), jnp.float32),
            jax.ShapeDtypeStruct((_N, _H), jnp.float32),
        ],
    )(x, wab)


def _make_gather(ne):
    epw = ne // _NW
    ch, nchunk = _chunking(epw)

    def _gather_body(ps_hbm, pr_hbm, is_hbm, ir_hbm, g_hbm,
                     idxs_v, idxr_v, rows,
                     sem_is, sem_ir, sem_g1, sem_g2, sem_w):
        c = lax.axis_index("c")
        s = lax.axis_index("s")
        base0 = (c * _NS + s) * epw

        def idx_fire(k, p):
            base = base0 + k * ch
            pltpu.async_copy(is_hbm.at[pl.ds(base, ch)], idxs_v.at[p], sem_is.at[p])
            pltpu.async_copy(ir_hbm.at[pl.ds(base, ch)], idxr_v.at[p], sem_ir.at[p])

        def idx_wait(p):
            pltpu.make_async_copy(is_hbm.at[pl.ds(base0, ch)], idxs_v.at[p], sem_is.at[p]).wait()
            pltpu.make_async_copy(ir_hbm.at[pl.ds(base0, ch)], idxr_v.at[p], sem_ir.at[p]).wait()

        def g1_fire(p):
            pltpu.async_copy(ps_hbm.at[idxs_v.at[p]], rows.at[p], sem_g1.at[p])

        def g1_wait(p):
            pltpu.make_async_copy(ps_hbm.at[idxs_v.at[p]], rows.at[p], sem_g1.at[p]).wait()

        def g2_fire(p):
            pltpu.async_copy(pr_hbm.at[idxr_v.at[p]], rows.at[p], sem_g2.at[p], add=True)

        def g2_wait(p):
            pltpu.make_async_copy(pr_hbm.at[idxr_v.at[p]], rows.at[p], sem_g2.at[p]).wait()

        def write_fire(k, p):
            base = base0 + k * ch
            pltpu.async_copy(rows.at[p], g_hbm.at[pl.ds(base, ch)], sem_w.at[p])

        def write_wait(p):
            pltpu.make_async_copy(rows.at[p], g_hbm.at[pl.ds(base0, ch)], sem_w.at[p]).wait()

        idx_fire(0, 0)
        idx_fire(1, 1)
        idx_fire(2, 2)
        idx_wait(0)
        g1_fire(0)

        def body(k, carry):
            p = lax.rem(k, 3)
            q = lax.rem(k + 1, 3)

            @pl.when(k + 1 < nchunk)
            def _():
                @pl.when(k >= 2)
                def _():
                    write_wait(q)
                idx_wait(q)
                g1_fire(q)

            g1_wait(p)
            g2_fire(p)
            g2_wait(p)
            write_fire(k, p)

            @pl.when(k + 3 < nchunk)
            def _():
                idx_fire(k + 3, p)

            return carry

        lax.fori_loop(0, nchunk, body, 0)
        write_wait(0)
        write_wait(1)
        write_wait(2)

    return pl.kernel(
        _gather_body,
        out_type=jax.ShapeDtypeStruct((ne, _H), jnp.float32),
        mesh=_mesh(),
        scratch_types=[
            pltpu.VMEM((3, ch), jnp.int32),
            pltpu.VMEM((3, ch), jnp.int32),
            pltpu.VMEM((3, ch, _H), jnp.float32),
            pltpu.SemaphoreType.DMA((3,)),
            pltpu.SemaphoreType.DMA((3,)),
            pltpu.SemaphoreType.DMA((3,)),
            pltpu.SemaphoreType.DMA((3,)),
            pltpu.SemaphoreType.DMA((3,)),
        ],
    )


_gather_edges0 = _make_gather(_EH0)
_gather_edges1 = _make_gather(_EH1)


def _mlp_body(g_ref, ea_ref, w1c_ref, b1_ref, w2_ref, b2_ref,
              w3_ref, b3_ref, out_ref):
    pre1 = (g_ref[...]
            + jnp.dot(ea_ref[...], w1c_ref[...], preferred_element_type=jnp.float32)
            + b1_ref[...])
    h = _silu(pre1)
    pre2 = jnp.dot(h, w2_ref[...], preferred_element_type=jnp.float32) + b2_ref[...]
    m = _silu(pre2)
    z = jnp.sum(m * w3_ref[...], axis=1, keepdims=True) + b3_ref[...]
    out_ref[...] = m * _sigmoid(z)


def _edge_mlp(g, ea, w1c, b1, w2, b2, w3row, b3):
    eb = 2560
    ne = g.shape[0]
    return pl.pallas_call(
        _mlp_body,
        grid=(ne // eb,),
        in_specs=[
            pl.BlockSpec((eb, _H), lambda i: (i, 0)),
            pl.BlockSpec((eb, _I), lambda i: (i, 0)),
            pl.BlockSpec((_I, _H), lambda i: (0, 0)),
            pl.BlockSpec((1, _H), lambda i: (0, 0)),
            pl.BlockSpec((_H, _H), lambda i: (0, 0)),
            pl.BlockSpec((1, _H), lambda i: (0, 0)),
            pl.BlockSpec((1, _H), lambda i: (0, 0)),
            pl.BlockSpec((1, 1), lambda i: (0, 0)),
        ],
        out_specs=pl.BlockSpec((eb, _H), lambda i: (i, 0)),
        out_shape=jax.ShapeDtypeStruct((ne, _H), jnp.float32),
    )(g, ea, w1c, b1, w2, b2, w3row, b3)


def _make_scatter(ne):
    epw = ne // _NW
    ch = 80
    nchunk = epw // ch

    def _scatter_body(w_hbm, ir_hbm, zeros_hbm, out_hbm, idx_v, rows_v, acc,
                      sem_i, sem_r, sem_s):
        c = lax.axis_index("c")
        s = lax.axis_index("s")
        row0 = s * _NROWS
        pltpu.sync_copy(zeros_hbm.at[pl.ds(row0, _NROWS)], acc.at[pl.ds(row0, _NROWS)])
        plsc.subcore_barrier()

        base0 = (c * _NS + s) * epw

        def load_fire(k, p):
            base = base0 + k * ch
            pltpu.async_copy(ir_hbm.at[pl.ds(base, ch)], idx_v.at[p], sem_i.at[p])
            pltpu.async_copy(w_hbm.at[pl.ds(base, ch)], rows_v.at[p], sem_r.at[p])

        def load_wait(p):
            pltpu.make_async_copy(ir_hbm.at[pl.ds(base0, ch)], idx_v.at[p], sem_i.at[p]).wait()
            pltpu.make_async_copy(w_hbm.at[pl.ds(base0, ch)], rows_v.at[p], sem_r.at[p]).wait()

        def scat_fire(p):
            pltpu.async_copy(rows_v.at[p], acc.at[idx_v.at[p]], sem_s.at[p], add=True)

        def scat_wait(p):
            pltpu.make_async_copy(rows_v.at[p], acc.at[idx_v.at[p]], sem_s.at[p]).wait()

        load_fire(0, 0)
        load_fire(1, 1)

        def body(k, carry):
            p = lax.rem(k, 3)
            r = lax.rem(k + 2, 3)
            load_wait(p)
            scat_fire(p)

            @pl.when(k + 2 < nchunk)
            def _():
                @pl.when(k >= 1)
                def _():
                    scat_wait(r)
                load_fire(k + 2, r)

            return carry

        lax.fori_loop(0, nchunk, body, 0)
        scat_wait(0)
        scat_wait(1)
        scat_wait(2)
        plsc.subcore_barrier()
        pltpu.sync_copy(acc.at[pl.ds(row0, _NROWS)], out_hbm.at[c, pl.ds(row0, _NROWS)])

    return pl.kernel(
        _scatter_body,
        out_type=jax.ShapeDtypeStruct((_NC, _NP, _H), jnp.float32),
        mesh=_mesh(),
        scratch_types=[
            pltpu.VMEM((3, ch), jnp.int32),
            pltpu.VMEM((3, ch, _H), jnp.float32),
            pltpu.VMEM_SHARED((_NP, _H), jnp.float32),
            pltpu.SemaphoreType.DMA((3,)),
            pltpu.SemaphoreType.DMA((3,)),
            pltpu.SemaphoreType.DMA((3,)),
        ],
    )


_scatter_edges0 = _make_scatter(_EH0)
_scatter_edges1 = _make_scatter(_EH1)


def _add_body(p0_ref, p1_ref, o_ref):
    o_ref[...] = (p0_ref[0] + p0_ref[1]) + (p1_ref[0] + p1_ref[1])


def _sum_partials(parts0, parts1):
    nb = 2000
    return pl.pallas_call(
        _add_body,
        grid=(_N // nb,),
        in_specs=[
            pl.BlockSpec((2, nb, _H), lambda i: (0, i, 0)),
            pl.BlockSpec((2, nb, _H), lambda i: (0, i, 0)),
        ],
        out_specs=pl.BlockSpec((nb, _H), lambda i: (i, 0)),
        out_shape=jax.ShapeDtypeStruct((_N, _H), jnp.float32),
    )(parts0, parts1)


def kernel(x, index, edge_attr, W1, b1, W2, b2, W3, b3):
    idx = index.astype(jnp.int32)
    idx_send = idx[0]
    idx_rec = idx[1]

    wab = jnp.stack([W1[:_H], W1[_H:2 * _H]])
    w1c = W1[2 * _H:]
    b1r = b1.reshape(1, _H)
    b2r = b2.reshape(1, _H)
    w3row = W3.reshape(1, _H)
    b3r = b3.reshape(1, 1)

    ps, pr = _project_nodes(x, wab)
    zeros = jnp.zeros((_NP, _H), jnp.float32)

    is0, is1 = idx_send[:_EH0], idx_send[_EH0:]
    ir0, ir1 = idx_rec[:_EH0], idx_rec[_EH0:]
    ea0, ea1 = edge_attr[:_EH0], edge_attr[_EH0:]

    g0 = _gather_edges0(ps, pr, is0, ir0)
    g1 = _gather_edges1(ps, pr, is1, ir1)
    w0 = _edge_mlp(g0, ea0, w1c, b1r, W2, b2r, w3row, b3r)
    w1 = _edge_mlp(g1, ea1, w1c, b1r, W2, b2r, w3row, b3r)
    parts0 = _scatter_edges0(w0, ir0, zeros)
    parts1 = _scatter_edges1(w1, ir1, zeros)
    return _sum_partials(parts0, parts1)


# speedup vs baseline: 1.7396x; 1.7396x over previous
import functools

import jax
import jax.numpy as jnp
from jax import lax
from jax.experimental import pallas as pl
from jax.experimental.pallas import tpu as pltpu
from jax.experimental.pallas import tpu_sc as plsc

_N = 10000
_E = 320000
_H = 128
_I = 16

_NC = 2
_NS = 16
_NW = _NC * _NS
_EH0 = 163840
_EH1 = _E - _EH0
_NP = 10240
_NROWS = _NP // _NS

_mesh = functools.partial(
    plsc.VectorSubcoreMesh, core_axis_name="c", subcore_axis_name="s"
)


def _silu(x):
    h = x * 0.5
    return h * (1.0 + jnp.tanh(h))


def _sigmoid(x):
    return 0.5 * (1.0 + jnp.tanh(x * 0.5))


def _chunking(epw):
    ch = 128 if epw % 128 == 0 else 80
    return ch, epw // ch


def _proj_body(x_ref, wab_ref, ps_ref, pr_ref):
    ps_ref[...] = jnp.dot(x_ref[0], wab_ref[0], preferred_element_type=jnp.float32)
    pr_ref[...] = jnp.dot(x_ref[1], wab_ref[1], preferred_element_type=jnp.float32)


def _project_nodes(x, wab):
    nb = 2000
    return pl.pallas_call(
        _proj_body,
        grid=(_N // nb,),
        in_specs=[
            pl.BlockSpec((2, nb, _H), lambda i: (0, i, 0)),
            pl.BlockSpec((2, _H, _H), lambda i: (0, 0, 0)),
        ],
        out_specs=[
            pl.BlockSpec((nb, _H), lambda i: (i, 0)),
            pl.BlockSpec((nb, _H), lambda i: (i, 0)),
        ],
        out_shape=[
            jax.ShapeDtypeStruct((_N, _H), jnp.float32),
            jax.ShapeDtypeStruct((_N, _H), jnp.float32),
        ],
    )(x, wab)


def _make_gather(ne):
    epw = ne // _NW
    ch, nchunk = _chunking(epw)

    def _gather_body(ps_hbm, pr_hbm, is_hbm, ir_hbm, g_hbm,
                     idxs_v, idxr_v, rows,
                     sem_is, sem_ir, sem_g1, sem_g2, sem_w):
        c = lax.axis_index("c")
        s = lax.axis_index("s")
        base0 = (c * _NS + s) * epw

        def idx_fire(k, p):
            base = base0 + k * ch
            pltpu.async_copy(is_hbm.at[pl.ds(base, ch)], idxs_v.at[p], sem_is.at[p])
            pltpu.async_copy(ir_hbm.at[pl.ds(base, ch)], idxr_v.at[p], sem_ir.at[p])

        def idx_wait(p):
            pltpu.make_async_copy(is_hbm.at[pl.ds(base0, ch)], idxs_v.at[p], sem_is.at[p]).wait()
            pltpu.make_async_copy(ir_hbm.at[pl.ds(base0, ch)], idxr_v.at[p], sem_ir.at[p]).wait()

        def g1_fire(p):
            pltpu.async_copy(ps_hbm.at[idxs_v.at[p]], rows.at[p], sem_g1.at[p])

        def g1_wait(p):
            pltpu.make_async_copy(ps_hbm.at[idxs_v.at[p]], rows.at[p], sem_g1.at[p]).wait()

        def g2_fire(p):
            pltpu.async_copy(pr_hbm.at[idxr_v.at[p]], rows.at[p], sem_g2.at[p], add=True)

        def g2_wait(p):
            pltpu.make_async_copy(pr_hbm.at[idxr_v.at[p]], rows.at[p], sem_g2.at[p]).wait()

        def write_fire(k, p):
            base = base0 + k * ch
            pltpu.async_copy(rows.at[p], g_hbm.at[pl.ds(base, ch)], sem_w.at[p])

        def write_wait(p):
            pltpu.make_async_copy(rows.at[p], g_hbm.at[pl.ds(base0, ch)], sem_w.at[p]).wait()

        idx_fire(0, 0)
        idx_fire(1, 1)
        idx_fire(2, 2)
        idx_wait(0)
        g1_fire(0)

        def body(k, carry):
            p = lax.rem(k, 3)
            q = lax.rem(k + 1, 3)

            @pl.when(k + 1 < nchunk)
            def _():
                @pl.when(k >= 2)
                def _():
                    write_wait(q)
                idx_wait(q)
                g1_fire(q)

            g1_wait(p)
            g2_fire(p)
            g2_wait(p)
            write_fire(k, p)

            @pl.when(k + 3 < nchunk)
            def _():
                idx_fire(k + 3, p)

            return carry

        lax.fori_loop(0, nchunk, body, 0)
        write_wait(0)
        write_wait(1)
        write_wait(2)

    return pl.kernel(
        _gather_body,
        out_type=jax.ShapeDtypeStruct((ne, _H), jnp.float32),
        mesh=_mesh(),
        scratch_types=[
            pltpu.VMEM((3, ch), jnp.int32),
            pltpu.VMEM((3, ch), jnp.int32),
            pltpu.VMEM((3, ch, _H), jnp.float32),
            pltpu.SemaphoreType.DMA((3,)),
            pltpu.SemaphoreType.DMA((3,)),
            pltpu.SemaphoreType.DMA((3,)),
            pltpu.SemaphoreType.DMA((3,)),
            pltpu.SemaphoreType.DMA((3,)),
        ],
    )


_gather_edges0 = _make_gather(_EH0)
_gather_edges1 = _make_gather(_EH1)


def _mlp_body(g_ref, eat_ref, w1c_ref, b1_ref, w2_ref, b2_ref,
              w3_ref, b3_ref, out_ref):
    ea_proj = lax.dot_general(
        eat_ref[...], w1c_ref[...],
        dimension_numbers=(((0,), (0,)), ((), ())),
        preferred_element_type=jnp.float32,
    )
    pre1 = g_ref[...] + ea_proj + b1_ref[...]
    h = _silu(pre1)
    pre2 = jnp.dot(h, w2_ref[...], preferred_element_type=jnp.float32) + b2_ref[...]
    m = _silu(pre2)
    z = jnp.sum(m * w3_ref[...], axis=1, keepdims=True) + b3_ref[...]
    out_ref[...] = m * _sigmoid(z)


def _edge_mlp(g, eat, w1c, b1, w2, b2, w3row, b3):
    ne = g.shape[0]
    eb = 4096 if ne % 4096 == 0 else 7808
    return pl.pallas_call(
        _mlp_body,
        grid=(ne // eb,),
        in_specs=[
            pl.BlockSpec((eb, _H), lambda i: (i, 0)),
            pl.BlockSpec((_I, eb), lambda i: (0, i)),
            pl.BlockSpec((_I, _H), lambda i: (0, 0)),
            pl.BlockSpec((1, _H), lambda i: (0, 0)),
            pl.BlockSpec((_H, _H), lambda i: (0, 0)),
            pl.BlockSpec((1, _H), lambda i: (0, 0)),
            pl.BlockSpec((1, _H), lambda i: (0, 0)),
            pl.BlockSpec((1, 1), lambda i: (0, 0)),
        ],
        out_specs=pl.BlockSpec((eb, _H), lambda i: (i, 0)),
        out_shape=jax.ShapeDtypeStruct((ne, _H), jnp.float32),
    )(g, eat, w1c, b1, w2, b2, w3row, b3)


def _make_scatter(ne):
    epw = ne // _NW
    ch = 80
    nchunk = epw // ch

    def _scatter_body(w_hbm, ir_hbm, zeros_hbm, out_hbm, idx_v, rows_v, acc,
                      sem_i, sem_r, sem_s):
        c = lax.axis_index("c")
        s = lax.axis_index("s")
        row0 = s * _NROWS
        pltpu.sync_copy(zeros_hbm.at[pl.ds(row0, _NROWS)], acc.at[pl.ds(row0, _NROWS)])
        plsc.subcore_barrier()

        base0 = (c * _NS + s) * epw

        def load_fire(k, p):
            base = base0 + k * ch
            pltpu.async_copy(ir_hbm.at[pl.ds(base, ch)], idx_v.at[p], sem_i.at[p])
            pltpu.async_copy(w_hbm.at[pl.ds(base, ch)], rows_v.at[p], sem_r.at[p])

        def load_wait(p):
            pltpu.make_async_copy(ir_hbm.at[pl.ds(base0, ch)], idx_v.at[p], sem_i.at[p]).wait()
            pltpu.make_async_copy(w_hbm.at[pl.ds(base0, ch)], rows_v.at[p], sem_r.at[p]).wait()

        def scat_fire(p):
            pltpu.async_copy(rows_v.at[p], acc.at[idx_v.at[p]], sem_s.at[p], add=True)

        def scat_wait(p):
            pltpu.make_async_copy(rows_v.at[p], acc.at[idx_v.at[p]], sem_s.at[p]).wait()

        load_fire(0, 0)
        load_fire(1, 1)

        def body(k, carry):
            p = lax.rem(k, 3)
            r = lax.rem(k + 2, 3)
            load_wait(p)
            scat_fire(p)

            @pl.when(k + 2 < nchunk)
            def _():
                @pl.when(k >= 1)
                def _():
                    scat_wait(r)
                load_fire(k + 2, r)

            return carry

        lax.fori_loop(0, nchunk, body, 0)
        scat_wait(0)
        scat_wait(1)
        scat_wait(2)
        plsc.subcore_barrier()
        pltpu.sync_copy(acc.at[pl.ds(row0, _NROWS)], out_hbm.at[c, pl.ds(row0, _NROWS)])

    return pl.kernel(
        _scatter_body,
        out_type=jax.ShapeDtypeStruct((_NC, _NP, _H), jnp.float32),
        mesh=_mesh(),
        scratch_types=[
            pltpu.VMEM((3, ch), jnp.int32),
            pltpu.VMEM((3, ch, _H), jnp.float32),
            pltpu.VMEM_SHARED((_NP, _H), jnp.float32),
            pltpu.SemaphoreType.DMA((3,)),
            pltpu.SemaphoreType.DMA((3,)),
            pltpu.SemaphoreType.DMA((3,)),
        ],
    )


_scatter_edges0 = _make_scatter(_EH0)
_scatter_edges1 = _make_scatter(_EH1)


def _add_body(p0_ref, p1_ref, o_ref):
    o_ref[...] = (p0_ref[0] + p0_ref[1]) + (p1_ref[0] + p1_ref[1])


def _sum_partials(parts0, parts1):
    nb = 2000
    return pl.pallas_call(
        _add_body,
        grid=(_N // nb,),
        in_specs=[
            pl.BlockSpec((2, nb, _H), lambda i: (0, i, 0)),
            pl.BlockSpec((2, nb, _H), lambda i: (0, i, 0)),
        ],
        out_specs=pl.BlockSpec((nb, _H), lambda i: (i, 0)),
        out_shape=jax.ShapeDtypeStruct((_N, _H), jnp.float32),
    )(parts0, parts1)


def kernel(x, index, edge_attr, W1, b1, W2, b2, W3, b3):
    idx = index.astype(jnp.int32)
    idx_send = idx[0]
    idx_rec = idx[1]

    wab = jnp.stack([W1[:_H], W1[_H:2 * _H]])
    w1c = W1[2 * _H:]
    b1r = b1.reshape(1, _H)
    b2r = b2.reshape(1, _H)
    w3row = W3.reshape(1, _H)
    b3r = b3.reshape(1, 1)

    ps, pr = _project_nodes(x, wab)
    zeros = jnp.zeros((_NP, _H), jnp.float32)

    is0, is1 = idx_send[:_EH0], idx_send[_EH0:]
    ir0, ir1 = idx_rec[:_EH0], idx_rec[_EH0:]
    eat = edge_attr.T
    eat0, eat1 = eat[:, :_EH0], eat[:, _EH0:]

    g0 = _gather_edges0(ps, pr, is0, ir0)
    g1 = _gather_edges1(ps, pr, is1, ir1)
    w0 = _edge_mlp(g0, eat0, w1c, b1r, W2, b2r, w3row, b3r)
    w1 = _edge_mlp(g1, eat1, w1c, b1r, W2, b2r, w3row, b3r)
    parts0 = _scatter_edges0(w0, ir0, zeros)
    parts1 = _scatter_edges1(w1, ir1, zeros)
    return _sum_partials(parts0, parts1)
